# Initial kernel scaffold; baseline (speedup 1.0000x reference)
import jax
import jax.numpy as jnp
from jax.experimental import pallas as pl


def kernel(p):
    raise NotImplementedError("write your pallas kernel here")


# speedup vs baseline: 7.2450x; 7.2450x over previous
import jax
import jax.numpy as jnp
from jax.experimental import pallas as pl

_K = 16
_MT = 512


def _knn_tile_kernel(p_ref, pt_ref, out_ref):
    n = p_ref.shape[2]
    pb = p_ref[0]
    xn = pb[0:1, :]
    yn = pb[1:2, :]
    zn = pb[2:3, :]
    p2 = xn * xn + yn * yn + zn * zn

    qx = pt_ref[0, :, 0:1]
    qy = pt_ref[0, :, 1:2]
    qz = pt_ref[0, :, 2:3]

    def b16(v):
        return v.astype(jnp.bfloat16).astype(jnp.float32)
    qp = b16(qx) * b16(xn) + b16(qy) * b16(yn) + b16(qz) * b16(zn)
    s = p2 - 2.0 * qp

    iota = jax.lax.broadcasted_iota(jnp.int32, (_MT, n), 1)
    inf = jnp.float32(jnp.inf)
    for k in range(_K):
        rowmin = jnp.min(s, axis=1, keepdims=True)
        cand = jnp.where(s == rowmin, iota, n)
        nstar = jnp.min(cand, axis=1, keepdims=True)
        sel = iota == nstar
        out_ref[0, 0, :, k:k + 1] = jnp.sum(jnp.where(sel, xn, 0.0), axis=1,
                                            keepdims=True)
        out_ref[0, 1, :, k:k + 1] = jnp.sum(jnp.where(sel, yn, 0.0), axis=1,
                                            keepdims=True)
        out_ref[0, 2, :, k:k + 1] = jnp.sum(jnp.where(sel, zn, 0.0), axis=1,
                                            keepdims=True)
        s = jnp.where(sel, inf, s)


def kernel(p):
    b, d, n = p.shape
    pt = jnp.swapaxes(p, 1, 2)
    grid = (b, n // _MT)
    return pl.pallas_call(
        _knn_tile_kernel,
        grid=grid,
        in_specs=[
            pl.BlockSpec((1, d, n), lambda i, j: (i, 0, 0)),
            pl.BlockSpec((1, _MT, d), lambda i, j: (i, j, 0)),
        ],
        out_specs=pl.BlockSpec((1, d, _MT, _K), lambda i, j: (i, 0, j, 0)),
        out_shape=jax.ShapeDtypeStruct((b, d, n, _K), p.dtype),
    )(p, pt)


# speedup vs baseline: 17.3276x; 2.3917x over previous
import dataclasses
import functools

import jax
import jax.numpy as jnp
from jax import lax
from jax.experimental import pallas as pl
from jax.experimental.pallas import tpu as pltpu
from jax.experimental.pallas import tpu_sc as plsc

_K = 16
_MT = 512
_CH = 4096


def _topk_tile_kernel(p_ref, pt_ref, idx_ref):
    n = p_ref.shape[2]
    pb = p_ref[0]
    xn = pb[0:1, :]
    yn = pb[1:2, :]
    zn = pb[2:3, :]
    p2 = xn * xn + yn * yn + zn * zn

    qx = pt_ref[0, :, 0:1]
    qy = pt_ref[0, :, 1:2]
    qz = pt_ref[0, :, 2:3]

    def b16(v):
        return v.astype(jnp.bfloat16).astype(jnp.float32)

    qp = b16(qx) * b16(xn) + b16(qy) * b16(yn) + b16(qz) * b16(zn)
    s = p2 - 2.0 * qp

    iota = lax.broadcasted_iota(jnp.int32, (_MT, n), 1)
    inf = jnp.float32(jnp.inf)
    for k in range(_K):
        rowmin = jnp.min(s, axis=1, keepdims=True)
        cand = jnp.where(s == rowmin, iota, n)
        nstar = jnp.min(cand, axis=1, keepdims=True)
        idx_ref[0, :, k:k + 1] = nstar
        if k + 1 < _K:
            s = jnp.where(iota == nstar, inf, s)


def _topk_indices(p, pt):
    b, d, n = p.shape
    return pl.pallas_call(
        _topk_tile_kernel,
        grid=(b, n // _MT),
        in_specs=[
            pl.BlockSpec((1, d, n), lambda i, j: (i, 0, 0)),
            pl.BlockSpec((1, _MT, d), lambda i, j: (i, j, 0)),
        ],
        out_specs=pl.BlockSpec((1, _MT, _K), lambda i, j: (i, j, 0)),
        out_shape=jax.ShapeDtypeStruct((b, n, _K), jnp.int32),
    )(p, pt)


def _sc_gather(p, idx_flat, b, d, n, mk):
    mesh = plsc.VectorSubcoreMesh(core_axis_name="c", subcore_axis_name="s")
    nworkers = 32
    per_w = mk // (nworkers // b)
    halves = nworkers // b

    cp = pltpu.CompilerParams()
    if "needs_layout_passes" in pltpu.CompilerParams.__dataclass_fields__:
        cp = dataclasses.replace(cp, needs_layout_passes=False)

    @functools.partial(
        pl.kernel,
        mesh=mesh,
        out_type=jax.ShapeDtypeStruct((b * d * mk,), jnp.float32),
        scratch_types=[
            pltpu.VMEM((n,), jnp.float32),
            pltpu.VMEM((n,), jnp.float32),
            pltpu.VMEM((n,), jnp.float32),
            pltpu.VMEM((_CH,), jnp.int32),
            pltpu.VMEM((_CH,), jnp.float32),
            pltpu.VMEM((_CH,), jnp.float32),
            pltpu.VMEM((_CH,), jnp.float32),
        ],
        compiler_params=cp,
    )
    def gather_kernel(p_hbm, idx_hbm, out_hbm,
                      tx, ty, tz, iv, ox, oy, oz):
        wid = lax.axis_index("s") * 2 + lax.axis_index("c")
        bb = wid // halves
        base = (wid % halves) * per_w
        pltpu.sync_copy(p_hbm.at[pl.ds((bb * 3 + 0) * n, n)], tx)
        pltpu.sync_copy(p_hbm.at[pl.ds((bb * 3 + 1) * n, n)], ty)
        pltpu.sync_copy(p_hbm.at[pl.ds((bb * 3 + 2) * n, n)], tz)

        @pl.loop(0, per_w, step=_CH)
        def _(c0):
            src = bb * mk + base + c0
            dst = (bb * 3) * mk + base + c0
            pltpu.sync_copy(idx_hbm.at[pl.ds(src, _CH)], iv)

            @pl.loop(0, _CH, step=16)
            def _(i):
                ivec = iv[pl.ds(i, 16)]
                ox[pl.ds(i, 16)] = plsc.load_gather(tx, [ivec])
                oy[pl.ds(i, 16)] = plsc.load_gather(ty, [ivec])
                oz[pl.ds(i, 16)] = plsc.load_gather(tz, [ivec])

            pltpu.sync_copy(ox, out_hbm.at[pl.ds(dst, _CH)])
            pltpu.sync_copy(oy, out_hbm.at[pl.ds(dst + mk, _CH)])
            pltpu.sync_copy(oz, out_hbm.at[pl.ds(dst + 2 * mk, _CH)])

    return gather_kernel(p, idx_flat)


def kernel(p):
    b, d, n = p.shape
    pt = jnp.swapaxes(p, 1, 2)
    idx = _topk_indices(p, pt)
    out = _sc_gather(p.reshape(b * d * n), idx.reshape(b * n * _K),
                     b, d, n, n * _K)
    return out.reshape(b, d, n, _K)


# speedup vs baseline: 28.8958x; 1.6676x over previous
import dataclasses
import functools

import jax
import jax.numpy as jnp
import numpy as np
from jax import lax
from jax.experimental import pallas as pl
from jax.experimental.pallas import tpu as pltpu
from jax.experimental.pallas import tpu_sc as plsc
from jax.experimental.shard_map import shard_map
from jax.sharding import Mesh, PartitionSpec as P

_K = 16
_MT = 512
_CH = 4096


def _topk_tile_kernel(p_ref, pt_ref, idx_ref):
    n = p_ref.shape[2]
    pb = p_ref[0]
    xn = pb[0:1, :]
    yn = pb[1:2, :]
    zn = pb[2:3, :]
    p2 = xn * xn + yn * yn + zn * zn

    qx = pt_ref[0, :, 0:1]
    qy = pt_ref[0, :, 1:2]
    qz = pt_ref[0, :, 2:3]

    def b16(v):
        return v.astype(jnp.bfloat16).astype(jnp.float32)

    qp = b16(qx) * b16(xn) + b16(qy) * b16(yn) + b16(qz) * b16(zn)
    s = p2 - 2.0 * qp

    iota = lax.broadcasted_iota(jnp.int32, (_MT, n), 1)
    inf = jnp.float32(jnp.inf)
    for k in range(_K):
        rowmin = jnp.min(s, axis=1, keepdims=True)
        cand = jnp.where(s == rowmin, iota, n)
        nstar = jnp.min(cand, axis=1, keepdims=True)
        idx_ref[0, :, k:k + 1] = nstar
        if k + 1 < _K:
            s = jnp.where(iota == nstar, inf, s)


def _topk_indices(p, pt):
    b, d, n = p.shape
    return pl.pallas_call(
        _topk_tile_kernel,
        grid=(b, n // _MT),
        in_specs=[
            pl.BlockSpec((1, d, n), lambda i, j: (i, 0, 0)),
            pl.BlockSpec((1, _MT, d), lambda i, j: (i, j, 0)),
        ],
        out_specs=pl.BlockSpec((1, _MT, _K), lambda i, j: (i, j, 0)),
        out_shape=jax.ShapeDtypeStruct((b, n, _K), jnp.int32),
    )(p, pt)


def _sc_gather(p, idx_flat, b, d, n, mk):
    mesh = plsc.VectorSubcoreMesh(core_axis_name="c", subcore_axis_name="s")
    nworkers = 32
    per_w = mk // (nworkers // b)
    halves = nworkers // b

    cp = pltpu.CompilerParams()
    if "needs_layout_passes" in pltpu.CompilerParams.__dataclass_fields__:
        cp = dataclasses.replace(cp, needs_layout_passes=False)

    @functools.partial(
        pl.kernel,
        mesh=mesh,
        out_type=jax.ShapeDtypeStruct((b * d * mk,), jnp.float32),
        scratch_types=[
            pltpu.VMEM((n,), jnp.float32),
            pltpu.VMEM((n,), jnp.float32),
            pltpu.VMEM((n,), jnp.float32),
            pltpu.VMEM((_CH,), jnp.int32),
            pltpu.VMEM((_CH,), jnp.float32),
            pltpu.VMEM((_CH,), jnp.float32),
            pltpu.VMEM((_CH,), jnp.float32),
        ],
        compiler_params=cp,
    )
    def gather_kernel(p_hbm, idx_hbm, out_hbm,
                      tx, ty, tz, iv, ox, oy, oz):
        wid = lax.axis_index("s") * 2 + lax.axis_index("c")
        bb = wid // halves
        base = (wid % halves) * per_w
        pltpu.sync_copy(p_hbm.at[pl.ds((bb * 3 + 0) * n, n)], tx)
        pltpu.sync_copy(p_hbm.at[pl.ds((bb * 3 + 1) * n, n)], ty)
        pltpu.sync_copy(p_hbm.at[pl.ds((bb * 3 + 2) * n, n)], tz)

        @pl.loop(0, per_w, step=_CH)
        def _(c0):
            src = bb * mk + base + c0
            dst = (bb * 3) * mk + base + c0
            pltpu.sync_copy(idx_hbm.at[pl.ds(src, _CH)], iv)

            @pl.loop(0, _CH, step=16)
            def _(i):
                ivec = iv[pl.ds(i, 16)]
                ox[pl.ds(i, 16)] = plsc.load_gather(tx, [ivec])
                oy[pl.ds(i, 16)] = plsc.load_gather(ty, [ivec])
                oz[pl.ds(i, 16)] = plsc.load_gather(tz, [ivec])

            pltpu.sync_copy(ox, out_hbm.at[pl.ds(dst, _CH)])
            pltpu.sync_copy(oy, out_hbm.at[pl.ds(dst + mk, _CH)])
            pltpu.sync_copy(oz, out_hbm.at[pl.ds(dst + 2 * mk, _CH)])

    return gather_kernel(p, idx_flat)


def _kernel_shard(p):
    b, d, n = p.shape
    pt = jnp.swapaxes(p, 1, 2)
    idx = _topk_indices(p, pt)
    out = _sc_gather(p.reshape(b * d * n), idx.reshape(b * n * _K),
                     b, d, n, n * _K)
    return out.reshape(b, d, n, _K)


def kernel(p):
    b = p.shape[0]
    devs = jax.devices()
    nd = 2 if (len(devs) >= 2 and b % 2 == 0) else 1
    mesh = Mesh(np.array(devs[:nd]), ("x",))
    f = shard_map(_kernel_shard, mesh=mesh,
                  in_specs=P("x"), out_specs=P("x"), check_rep=False)
    return f(p)


# speedup vs baseline: 32.0581x; 1.1094x over previous
import dataclasses
import functools

import jax
import jax.numpy as jnp
import numpy as np
from jax import lax
from jax.experimental import pallas as pl
from jax.experimental.pallas import tpu as pltpu
from jax.experimental.pallas import tpu_sc as plsc
from jax.experimental.shard_map import shard_map
from jax.sharding import Mesh, PartitionSpec as P

_K = 16
_MT = 512
_CH = 4096


def _topk_tile_kernel(p_ref, pt_ref, idx_ref):
    n = p_ref.shape[2]
    pb = p_ref[0]
    xn = pb[0:1, :]
    yn = pb[1:2, :]
    zn = pb[2:3, :]
    p2 = xn * xn + yn * yn + zn * zn

    qx = pt_ref[0, :, 0:1]
    qy = pt_ref[0, :, 1:2]
    qz = pt_ref[0, :, 2:3]

    def b16(v):
        return v.astype(jnp.bfloat16).astype(jnp.float32)

    qp = b16(qx) * b16(xn) + b16(qy) * b16(yn) + b16(qz) * b16(zn)
    s = p2 - 2.0 * qp

    iota = lax.broadcasted_iota(jnp.int32, (_MT, n), 1)
    inf = jnp.float32(jnp.inf)
    for k in range(_K):
        nstar = jnp.argmin(s, axis=1).astype(jnp.int32)
        nstar = nstar[:, None]
        idx_ref[0, :, k:k + 1] = nstar
        if k + 1 < _K:
            s = jnp.where(iota == nstar, inf, s)


def _topk_indices(p, pt):
    b, d, n = p.shape
    return pl.pallas_call(
        _topk_tile_kernel,
        grid=(b, n // _MT),
        in_specs=[
            pl.BlockSpec((1, d, n), lambda i, j: (i, 0, 0)),
            pl.BlockSpec((1, _MT, d), lambda i, j: (i, j, 0)),
        ],
        out_specs=pl.BlockSpec((1, _MT, _K), lambda i, j: (i, j, 0)),
        out_shape=jax.ShapeDtypeStruct((b, n, _K), jnp.int32),
    )(p, pt)


def _sc_gather(p, idx_flat, b, d, n, mk):
    mesh = plsc.VectorSubcoreMesh(core_axis_name="c", subcore_axis_name="s")
    nworkers = 32
    per_w = mk // (nworkers // b)
    halves = nworkers // b

    cp = pltpu.CompilerParams()
    if "needs_layout_passes" in pltpu.CompilerParams.__dataclass_fields__:
        cp = dataclasses.replace(cp, needs_layout_passes=False)

    @functools.partial(
        pl.kernel,
        mesh=mesh,
        out_type=jax.ShapeDtypeStruct((b * d * mk,), jnp.float32),
        scratch_types=[
            pltpu.VMEM((n,), jnp.float32),
            pltpu.VMEM((n,), jnp.float32),
            pltpu.VMEM((n,), jnp.float32),
            pltpu.VMEM((_CH,), jnp.int32),
            pltpu.VMEM((_CH,), jnp.float32),
            pltpu.VMEM((_CH,), jnp.float32),
            pltpu.VMEM((_CH,), jnp.float32),
        ],
        compiler_params=cp,
    )
    def gather_kernel(p_hbm, idx_hbm, out_hbm,
                      tx, ty, tz, iv, ox, oy, oz):
        wid = lax.axis_index("s") * 2 + lax.axis_index("c")
        bb = wid // halves
        base = (wid % halves) * per_w
        pltpu.sync_copy(p_hbm.at[pl.ds((bb * 3 + 0) * n, n)], tx)
        pltpu.sync_copy(p_hbm.at[pl.ds((bb * 3 + 1) * n, n)], ty)
        pltpu.sync_copy(p_hbm.at[pl.ds((bb * 3 + 2) * n, n)], tz)

        @pl.loop(0, per_w, step=_CH)
        def _(c0):
            src = bb * mk + base + c0
            dst = (bb * 3) * mk + base + c0
            pltpu.sync_copy(idx_hbm.at[pl.ds(src, _CH)], iv)

            @pl.loop(0, _CH, step=16)
            def _(i):
                ivec = iv[pl.ds(i, 16)]
                ox[pl.ds(i, 16)] = plsc.load_gather(tx, [ivec])
                oy[pl.ds(i, 16)] = plsc.load_gather(ty, [ivec])
                oz[pl.ds(i, 16)] = plsc.load_gather(tz, [ivec])

            pltpu.sync_copy(ox, out_hbm.at[pl.ds(dst, _CH)])
            pltpu.sync_copy(oy, out_hbm.at[pl.ds(dst + mk, _CH)])
            pltpu.sync_copy(oz, out_hbm.at[pl.ds(dst + 2 * mk, _CH)])

    return gather_kernel(p, idx_flat)


def _kernel_shard(p):
    b, d, n = p.shape
    pt = jnp.swapaxes(p, 1, 2)
    idx = _topk_indices(p, pt)
    out = _sc_gather(p.reshape(b * d * n), idx.reshape(b * n * _K),
                     b, d, n, n * _K)
    return out.reshape(b, d, n, _K)


def kernel(p):
    b = p.shape[0]
    devs = jax.devices()
    nd = 2 if (len(devs) >= 2 and b % 2 == 0) else 1
    mesh = Mesh(np.array(devs[:nd]), ("x",))
    f = shard_map(_kernel_shard, mesh=mesh,
                  in_specs=P("x"), out_specs=P("x"), check_rep=False)
    return f(p)
